# baseline (device time: 28934 ns/iter reference)
import jax
import jax.numpy as jnp
from jax import lax
from jax.experimental import pallas as pl
from jax.experimental.pallas import tpu as pltpu

N_DEV = 32
C = 16


def kernel(x, router_W, route_idx, expert_W):
    n, d = x.shape
    n_exp = router_W.shape[1]
    e_local = expert_W.shape[0]
    h = expert_W.shape[2]

    def body(x_ref, rw_ref, idx_ref, ew_ref, out_ref,
             stage_ref, recv_ref, gather_ref, send_sems, recv_sems):
        my = lax.axis_index("i")

        barrier = pltpu.get_barrier_semaphore()
        for r in range(1, N_DEV):
            pl.semaphore_signal(barrier, inc=1,
                                device_id=(lax.rem(my + r, N_DEV),),
                                device_id_type=pl.DeviceIdType.MESH)

        xf = x_ref[:, :]
        scores = jnp.dot(xf, rw_ref[:, :],
                         preferred_element_type=jnp.float32)
        smax = jnp.max(scores, axis=1, keepdims=True)
        p = jnp.exp(scores - smax)

        idx0 = idx_ref[:, 0:1]
        idx1 = idx_ref[:, 1:2]
        cols = lax.broadcasted_iota(jnp.int32, (n, n_exp), 1)
        g0 = jnp.sum(jnp.where(cols == idx0, p, 0.0), axis=1,
                     keepdims=True)
        g1 = jnp.sum(jnp.where(cols == idx1, p, 0.0), axis=1,
                     keepdims=True)
        gs = g0 + g1

        gated = []
        for j in range(e_local):
            e = my * e_local + j
            p_e = jnp.sum(jnp.where(cols == e, p, 0.0), axis=1,
                          keepdims=True)
            mask = jnp.logical_or(idx0 == e, idx1 == e)
            g_e = jnp.where(mask, p_e / gs, 0.0)
            gated.append((xf * g_e).astype(jnp.bfloat16))
        xg = jnp.concatenate(gated, axis=1)
        w = jnp.concatenate([ew_ref[j].astype(jnp.bfloat16)
                             for j in range(e_local)], axis=0)
        acc = jnp.dot(xg, w, preferred_element_type=jnp.float32)
        stage_ref[:, :] = acc.astype(jnp.bfloat16)

        pl.semaphore_wait(barrier, N_DEV - 1)

        rs = []
        for r in range(1, N_DEV):
            t = lax.rem(my + r, N_DEV)
            rdma = pltpu.make_async_remote_copy(
                src_ref=stage_ref.at[pl.ds(t * C, C)],
                dst_ref=recv_ref.at[r - 1],
                send_sem=send_sems.at[r - 1],
                recv_sem=recv_sems.at[r - 1],
                device_id=(t,),
                device_id_type=pl.DeviceIdType.MESH,
            )
            rdma.start()
            rs.append(rdma)

        for rdma in rs:
            rdma.wait_recv()
        red = (
            stage_ref[pl.ds(my * C, C), :].astype(jnp.float32)
            + jnp.sum(recv_ref[:, :, :].astype(jnp.float32), axis=0)
        )
        gather_ref[pl.ds(my * C, C), :] = red.astype(jnp.bfloat16)

        ag = []
        for r in range(1, N_DEV):
            t = lax.rem(my + r, N_DEV)
            rdma = pltpu.make_async_remote_copy(
                src_ref=gather_ref.at[pl.ds(my * C, C)],
                dst_ref=gather_ref.at[pl.ds(my * C, C)],
                send_sem=send_sems.at[N_DEV - 1 + r - 1],
                recv_sem=recv_sems.at[N_DEV - 1 + r - 1],
                device_id=(t,),
                device_id_type=pl.DeviceIdType.MESH,
            )
            rdma.start()
            ag.append(rdma)

        for rdma in ag:
            rdma.wait_recv()
        out_ref[:, :] = gather_ref[:, :].astype(jnp.float32)

        for rdma in rs:
            rdma.wait_send()
        for rdma in ag:
            rdma.wait_send()

    return pl.pallas_call(
        body,
        out_shape=jax.ShapeDtypeStruct((n, h), jnp.float32),
        in_specs=[pl.BlockSpec(memory_space=pltpu.VMEM)] * 4,
        out_specs=pl.BlockSpec(memory_space=pltpu.VMEM),
        scratch_shapes=[
            pltpu.VMEM((n, h), jnp.bfloat16),
            pltpu.VMEM((N_DEV - 1, C, h), jnp.bfloat16),
            pltpu.VMEM((n, h), jnp.bfloat16),
            pltpu.SemaphoreType.DMA((2 * (N_DEV - 1),)),
            pltpu.SemaphoreType.DMA((2 * (N_DEV - 1),)),
        ],
        compiler_params=pltpu.CompilerParams(collective_id=0),
    )(x, router_W, route_idx, expert_W)
